# baseline (speedup 1.0000x reference)
import functools

import jax
import jax.numpy as jnp
from jax import lax
from jax.experimental import pallas as pl
from jax.experimental.pallas import tpu as pltpu
from jax.experimental.pallas import tpu_sc as plsc

_T = 32
_PJ = 32
_ROWS = _T * _PJ
_KSUB = _ROWS // 128
_REAL_POS = list(range(16)) + list(range(21, 31))


def _sc_embedding_bag(xr, table):
    info = plsc.get_sparse_core_info()
    nw = info.num_cores * info.num_subcores
    n = xr.shape[0]
    d = table.shape[1]
    cpw = n // (_T * nw)

    @functools.partial(
        pl.kernel,
        out_type=jax.ShapeDtypeStruct((n, d), jnp.float32),
        mesh=plsc.VectorSubcoreMesh(core_axis_name="c", subcore_axis_name="s"),
        compiler_params=pltpu.CompilerParams(use_tc_tiling_on_sc=False),
        scratch_types=[
            pltpu.VMEM((_T, 33), jnp.float32),
            pltpu.VMEM((_KSUB, 128), jnp.int32),
            pltpu.VMEM((_ROWS, d), jnp.float32),
            pltpu.VMEM((_T, d), jnp.float32),
            pltpu.SemaphoreType.DMA,
        ],
    )
    def k(x_h, table_h, out_h, x_v, idx_v, rows_v, emb_v, sem):
        wid = lax.axis_index("c") * info.num_subcores + lax.axis_index("s")
        lane = lax.iota(jnp.int32, 16)
        live_hi = jnp.logical_and(lane >= 5, lane <= 14)

        def chunk_body(c, carry):
            base = (wid * cpw + c) * _T
            pltpu.sync_copy(x_h.at[pl.ds(base, _T)], x_v)

            def idx_body(t, carry2):
                lo = x_v[t, pl.ds(6, 16)].astype(jnp.int32)
                hi = x_v[t, pl.ds(17, 16)].astype(jnp.int32)
                hi = jnp.where(live_hi, hi, 0)
                r = t // 4
                col = (t % 4) * _PJ
                idx_v[r, pl.ds(col, 16)] = lo
                idx_v[r, pl.ds(col + 16, 16)] = hi
                return carry2

            lax.fori_loop(0, _T, idx_body, 0)

            copies = [
                pltpu.async_copy(
                    table_h.at[idx_v.at[kk]],
                    rows_v.at[pl.ds(kk * 128, 128)],
                    sem,
                )
                for kk in range(_KSUB)
            ]
            for cp in copies:
                cp.wait()

            def tok_body(t, carry2):
                for q in range(d // 16):
                    a = rows_v[t * _PJ, pl.ds(q * 16, 16)]
                    for p in _REAL_POS[1:]:
                        a = a + rows_v[t * _PJ + p, pl.ds(q * 16, 16)]
                    emb_v[t, pl.ds(q * 16, 16)] = a
                return carry2

            lax.fori_loop(0, _T, tok_body, 0)
            pltpu.sync_copy(emb_v, out_h.at[pl.ds(base, _T)])
            return carry

        lax.fori_loop(0, cpw, chunk_body, 0)

    return k(xr, table)


def _tc_dense(xr, emb, row0, nw1, nb1, nw2, nb2, tw1, tb1, tw2, tb2,
              admv, gv, bv):
    n, c = xr.shape
    d = emb.shape[1]
    bt = 1024
    grid = n // bt

    def body(x_ref, e_ref, r0_r, nw1_r, nb1_r, nw2_r, nb2_r, tw1_r, tb1_r,
             tw2_r, tb2_r, adm_r, g_r, b_r, o_ref):
        xs = x_ref[...]
        ci = xs[:, 6:32]
        cnt = jnp.sum(jnp.where(ci == 0.0, 1.0, 0.0), axis=1, keepdims=True)
        emb_b = e_ref[...] - cnt * r0_r[...]

        num = xs[:, 5:6]
        nmask = jnp.isnan(num)
        numc = jnp.where(nmask, 0.0, num)
        h1 = jnp.maximum(numc * nw1_r[...] + nb1_r[...], 0.0)
        no = jnp.dot(h1, nw2_r[...], preferred_element_type=jnp.float32)
        no = jnp.where(nmask, 0.0, no + nb2_r[...])

        td = xs[:, 0:5]
        tmask = jnp.isnan(td[:, 0:1])
        tdc = jnp.where(jnp.isnan(td), 0.0, td)
        acc = tb1_r[...]
        for kk in range(5):
            acc = acc + tdc[:, kk:kk + 1] * tw1_r[kk:kk + 1, :]
        h2 = jnp.maximum(acc, 0.0)
        to = jnp.dot(h2, tw2_r[...], preferred_element_type=jnp.float32)
        to = jnp.where(tmask, 0.0, to + tb2_r[...])

        enc = emb_b + no + to
        nrm = jnp.sqrt(jnp.sum(enc * enc, axis=1, keepdims=True))
        enc = enc / jnp.maximum(nrm, 1e-10)
        enc = enc + xs[:, 32:33] * adm_r[...]
        mu = jnp.mean(enc, axis=1, keepdims=True)
        dev = enc - mu
        var = jnp.mean(dev * dev, axis=1, keepdims=True)
        o_ref[...] = dev * lax.rsqrt(var + 1e-5) * g_r[...] + b_r[...]

    full = lambda shape: pl.BlockSpec(shape, lambda i: (0, 0))
    return pl.pallas_call(
        body,
        grid=(grid,),
        in_specs=[
            pl.BlockSpec((bt, c), lambda i: (i, 0)),
            pl.BlockSpec((bt, d), lambda i: (i, 0)),
            full((1, d)),
            full((1, d)), full((1, d)), full((d, d)), full((1, d)),
            full((5, d)), full((1, d)), full((d, d)), full((1, d)),
            full((1, d)), full((1, d)), full((1, d)),
        ],
        out_specs=pl.BlockSpec((bt, d), lambda i: (i, 0)),
        out_shape=jax.ShapeDtypeStruct((n, d), jnp.float32),
    )(xr, emb, row0, nw1, nb1, nw2, nb2, tw1, tb1, tw2, tb2, admv, gv, bv)


def kernel(x, table, nw1, nb1, nw2, nb2, tw1, tb1, tw2, tb2, adm, gamma, beta):
    b, s, c = x.shape
    n = b * s
    d = table.shape[1]
    xr = x.reshape(n, c)
    emb = _sc_embedding_bag(xr, table)
    out = _tc_dense(
        xr, emb, table[0:1],
        nw1, nb1.reshape(1, d), nw2, nb2.reshape(1, d),
        tw1, tb1.reshape(1, d), tw2, tb2.reshape(1, d),
        adm.reshape(1, d), gamma.reshape(1, d), beta.reshape(1, d),
    )
    return out.reshape(b, s, d)


# speedup vs baseline: 6.6801x; 6.6801x over previous
import functools

import jax
import jax.numpy as jnp
from jax import lax
from jax.experimental import pallas as pl
from jax.experimental.pallas import tpu as pltpu
from jax.experimental.pallas import tpu_sc as plsc

_T = 128
_J = 26
_HJ = 13


def _tc_index_prep(xr):
    n, c = xr.shape
    pc = 4
    bt = pc * _T
    grid = n // bt

    def body(x_ref, o_ref):
        ci = x_ref[:, 6:32].astype(jnp.int32)
        cit = ci.T
        for cc in range(pc):
            o_ref[cc * _J:(cc + 1) * _J, :] = cit[:, cc * _T:(cc + 1) * _T]

    return pl.pallas_call(
        body,
        grid=(grid,),
        in_specs=[pl.BlockSpec((bt, c), lambda i: (i, 0))],
        out_specs=pl.BlockSpec((pc * _J, 128), lambda i: (i, 0)),
        out_shape=jax.ShapeDtypeStruct((n // _T * _J, 128), jnp.int32),
    )(xr)


def _sc_embedding_bag(idx2, table, n_tokens):
    info = plsc.get_sparse_core_info()
    nw = info.num_cores * info.num_subcores
    d = table.shape[1]
    g_total = n_tokens // _T
    _HT = _T // 2

    @functools.partial(
        pl.kernel,
        out_type=jax.ShapeDtypeStruct((n_tokens, d), jnp.float32),
        mesh=plsc.VectorSubcoreMesh(core_axis_name="c", subcore_axis_name="s"),
        compiler_params=pltpu.CompilerParams(use_tc_tiling_on_sc=False),
        scratch_types=[
            pltpu.VMEM((_J, 128), jnp.int32),
            pltpu.VMEM((_HJ * _HT, d), jnp.float32),
            pltpu.VMEM((_HJ * _HT, d), jnp.float32),
            pltpu.VMEM((_T, d), jnp.float32),
            pltpu.SemaphoreType.DMA,
            pltpu.SemaphoreType.DMA,
        ],
    )
    def k(idx_h, table_h, out_h, idx_v, rows_x, rows_y, emb_v, sem_x, sem_y):
        wid = lax.axis_index("c") * info.num_subcores + lax.axis_index("s")
        extra = jnp.where(wid < 16, 1, 0)
        cpw = 12 + extra
        base_chunk = jnp.where(wid < 16, wid * 13, 208 + (wid - 16) * 12)

        def issue(buf, sem, h, p):
            return [
                pltpu.async_copy(
                    table_h.at[idx_v.at[p * _HJ + j, pl.ds(h * _HT, _HT)]],
                    buf.at[pl.ds(j * _HT, _HT)],
                    sem,
                )
                for j in range(_HJ)
            ]

        def accumulate(buf, h, p):
            def tok(t, carry2):
                for q in range(d // 16):
                    if p == 0:
                        a = buf[t, pl.ds(q * 16, 16)]
                        jj = range(1, _HJ)
                    else:
                        a = emb_v[h * _HT + t, pl.ds(q * 16, 16)]
                        jj = range(_HJ)
                    for j in jj:
                        a = a + buf[j * _HT + t, pl.ds(q * 16, 16)]
                    emb_v[h * _HT + t, pl.ds(q * 16, 16)] = a
                return carry2

            lax.fori_loop(0, _HT, tok, 0)

        def chunk_body(c, carry):
            g = base_chunk + c
            pltpu.sync_copy(idx_h.at[pl.ds(g * _J, _J)], idx_v)
            cps0 = issue(rows_x, sem_x, 0, 0)
            cps1 = issue(rows_y, sem_y, 0, 1)
            for cp in cps0:
                cp.wait()
            accumulate(rows_x, 0, 0)
            cps2 = issue(rows_x, sem_x, 1, 0)
            for cp in cps1:
                cp.wait()
            accumulate(rows_y, 0, 1)
            cps3 = issue(rows_y, sem_y, 1, 1)
            for cp in cps2:
                cp.wait()
            accumulate(rows_x, 1, 0)
            for cp in cps3:
                cp.wait()
            accumulate(rows_y, 1, 1)
            pltpu.sync_copy(emb_v, out_h.at[pl.ds(g * _T, _T)])
            return carry

        lax.fori_loop(0, cpw, chunk_body, 0)

    return k(idx2, table)


def _tc_dense(xr, emb, row0, nw1, nb1, nw2, nb2, tw1, tb1, tw2, tb2,
              admv, gv, bv):
    n, c = xr.shape
    d = emb.shape[1]
    bt = 1024
    grid = n // bt

    def body(x_ref, e_ref, r0_r, nw1_r, nb1_r, nw2_r, nb2_r, tw1_r, tb1_r,
             tw2_r, tb2_r, adm_r, g_r, b_r, o_ref):
        xs = x_ref[...]
        ci = xs[:, 6:32]
        cnt = jnp.sum(jnp.where(ci == 0.0, 1.0, 0.0), axis=1, keepdims=True)
        emb_b = e_ref[...] - cnt * r0_r[...]

        num = xs[:, 5:6]
        nmask = jnp.isnan(num)
        numc = jnp.where(nmask, 0.0, num)
        h1 = jnp.maximum(numc * nw1_r[...] + nb1_r[...], 0.0)
        no = jnp.dot(h1, nw2_r[...], preferred_element_type=jnp.float32)
        no = jnp.where(nmask, 0.0, no + nb2_r[...])

        td = xs[:, 0:5]
        tmask = jnp.isnan(td[:, 0:1])
        tdc = jnp.where(jnp.isnan(td), 0.0, td)
        acc = tb1_r[...]
        for kk in range(5):
            acc = acc + tdc[:, kk:kk + 1] * tw1_r[kk:kk + 1, :]
        h2 = jnp.maximum(acc, 0.0)
        to = jnp.dot(h2, tw2_r[...], preferred_element_type=jnp.float32)
        to = jnp.where(tmask, 0.0, to + tb2_r[...])

        enc = emb_b + no + to
        nrm = jnp.sqrt(jnp.sum(enc * enc, axis=1, keepdims=True))
        enc = enc / jnp.maximum(nrm, 1e-10)
        enc = enc + xs[:, 32:33] * adm_r[...]
        mu = jnp.mean(enc, axis=1, keepdims=True)
        dev = enc - mu
        var = jnp.mean(dev * dev, axis=1, keepdims=True)
        o_ref[...] = dev * lax.rsqrt(var + 1e-5) * g_r[...] + b_r[...]

    full = lambda shape: pl.BlockSpec(shape, lambda i: (0, 0))
    return pl.pallas_call(
        body,
        grid=(grid,),
        in_specs=[
            pl.BlockSpec((bt, c), lambda i: (i, 0)),
            pl.BlockSpec((bt, d), lambda i: (i, 0)),
            full((1, d)),
            full((1, d)), full((1, d)), full((d, d)), full((1, d)),
            full((5, d)), full((1, d)), full((d, d)), full((1, d)),
            full((1, d)), full((1, d)), full((1, d)),
        ],
        out_specs=pl.BlockSpec((bt, d), lambda i: (i, 0)),
        out_shape=jax.ShapeDtypeStruct((n, d), jnp.float32),
    )(xr, emb, row0, nw1, nb1, nw2, nb2, tw1, tb1, tw2, tb2, admv, gv, bv)


def kernel(x, table, nw1, nb1, nw2, nb2, tw1, tb1, tw2, tb2, adm, gamma, beta):
    b, s, c = x.shape
    n = b * s
    d = table.shape[1]
    xr = x.reshape(n, c)
    idx2 = _tc_index_prep(xr)
    emb = _sc_embedding_bag(idx2, table, n)
    out = _tc_dense(
        xr, emb, table[0:1],
        nw1, nb1.reshape(1, d), nw2, nb2.reshape(1, d),
        tw1, tb1.reshape(1, d), tw2, tb2.reshape(1, d),
        adm.reshape(1, d), gamma.reshape(1, d), beta.reshape(1, d),
    )
    return out.reshape(b, s, d)
